# baseline (speedup 1.0000x reference)
import functools

import jax
import jax.numpy as jnp
from jax import lax
from jax.experimental import pallas as pl
from jax.experimental.pallas import tpu as pltpu
from jax.experimental.pallas import tpu_sc as plsc

_S, _D = 8192, 768
_H, _DH = 12, 64
_NHR = 2
_NB = 256
_CHUNK = 64
_NCH = _S * _NHR // _CHUNK
_SEG = _H * _NHR
_SB = 512

_INTERPRET = False


def _proj_hash_body(h_ref, wqk_ref, wv_ref, rot_ref, eye_ref, qkv_ref, bkt_ref):
    hb = h_ref[...]
    wqk = wqk_ref[...]
    wv = wv_ref[...]
    eye = eye_ref[...]
    qk = lax.dot_general(hb, wqk, (((1,), (1,)), ((), ())),
                         preferred_element_type=jnp.float32)
    v = lax.dot_general(hb, wv, (((1,), (1,)), ((), ())),
                        preferred_element_type=jnp.float32)
    pieces = []
    rot = rot_ref[...]
    i2 = lax.broadcasted_iota(jnp.int32, (_SB, 128), 1)
    for h in range(_H):
        qh = qk[:, h * _DH:(h + 1) * _DH]
        pieces.append(qh)
        pieces.append(v[:, h * _DH:(h + 1) * _DH])
        r = lax.dot_general(qh, rot, (((1,), (0,)), ((), ())),
                            preferred_element_type=jnp.float32)
        for n in range(_NHR):
            rr = r[:, n * 128:(n + 1) * 128]
            mx = jnp.max(jnp.abs(rr), axis=1, keepdims=True)
            t1 = jnp.where(rr == mx, i2, 2 * _NB)
            t2 = jnp.where(rr == -mx, i2 + 128, 2 * _NB)
            am = jnp.min(jnp.minimum(t1, t2), axis=1,
                         keepdims=True).astype(jnp.float32)
            am_row = lax.dot_general(am, eye, (((0,), (0,)), ((), ())),
                                     preferred_element_type=jnp.float32)
            c = h * _NHR + n
            bkt_ref[c:c + 1, :] = am_row
    qkv_ref[...] = jnp.concatenate(pieces, axis=1)


def _proj_hash(h2, wqk, wv, rot2):
    grid = _S // _SB
    eye = jnp.eye(_SB, dtype=jnp.float32)
    return pl.pallas_call(
        _proj_hash_body,
        grid=(grid,),
        in_specs=[
            pl.BlockSpec((_SB, _D), lambda i: (i, 0)),
            pl.BlockSpec((_H * _DH, _D), lambda i: (0, 0)),
            pl.BlockSpec((_H * _DH, _D), lambda i: (0, 0)),
            pl.BlockSpec((_DH, _NHR * 128), lambda i: (0, 0)),
            pl.BlockSpec((_SB, _SB), lambda i: (0, 0)),
        ],
        out_specs=[
            pl.BlockSpec((_SB, _H * 2 * _DH), lambda i: (i, 0)),
            pl.BlockSpec((_SEG, _SB), lambda i: (0, i)),
        ],
        out_shape=[
            jax.ShapeDtypeStruct((_S, _H * 2 * _DH), jnp.float32),
            jax.ShapeDtypeStruct((_SEG, _S), jnp.float32),
        ],
        interpret=_INTERPRET,
    )(h2, wqk, wv, rot2, eye)


_NT = _S // 128


def _sort_body(bkt_ref, tri_ref, low_ref, dest_ref):
    triu = tri_ref[...]
    low = low_ref[...]

    beta = lax.broadcasted_iota(jnp.int32, (_SEG, _NB, 128), 1)

    def onehot(t):
        b = bkt_ref[:, pl.ds(t * 128, 128)].astype(jnp.int32)
        return (b[:, None, :] == beta).astype(jnp.float32)

    def p1_body(t, carry):
        return carry + jnp.sum(onehot(t), axis=2)
    tot = lax.fori_loop(0, _NT, p1_body,
                        jnp.zeros((_SEG, _NB), jnp.float32), unroll=4)

    start = lax.dot_general(tot, low, (((1,), (0,)), ((), ())),
                            preferred_element_type=jnp.float32)

    def p2_body(t, pre):
        oh3 = onehot(t)
        oh2 = oh3.reshape(_SEG * _NB, 128)
        cnt = lax.dot_general(oh2, triu, (((1,), (0,)), ((), ())),
                              preferred_element_type=jnp.float32)
        cnt3 = cnt.reshape(_SEG, _NB, 128)
        d3 = oh3 * (cnt3 + (pre + start)[:, :, None])
        dest = jnp.sum(d3, axis=1) - 1.0
        dest_ref[:, :, pl.ds(t * 128, 128)] = dest[:, None, :].astype(jnp.int32)
        return pre + jnp.sum(oh3, axis=2)
    lax.fori_loop(0, _NT, p2_body, jnp.zeros((_SEG, _NB), jnp.float32),
                  unroll=4)


def _sort_dest(bkt):
    r = lax.broadcasted_iota(jnp.int32, (128, 128), 0)
    c = lax.broadcasted_iota(jnp.int32, (128, 128), 1)
    triu = (r <= c).astype(jnp.float32)
    r2 = lax.broadcasted_iota(jnp.int32, (_NB, _NB), 0)
    c2 = lax.broadcasted_iota(jnp.int32, (_NB, _NB), 1)
    low = (r2 < c2).astype(jnp.float32)
    return pl.pallas_call(
        _sort_body,
        grid=(1,),
        in_specs=[pl.BlockSpec((_SEG, _S), lambda i: (0, 0)),
                  pl.BlockSpec((128, 128), lambda i: (0, 0)),
                  pl.BlockSpec((_NB, _NB), lambda i: (0, 0))],
        out_specs=pl.BlockSpec((_SEG, 1, _S), lambda i: (0, 0, 0)),
        out_shape=jax.ShapeDtypeStruct((_SEG, 1, _S), jnp.int32),
        interpret=_INTERPRET,
    )(bkt, triu, low)


_GCH = 32
_NG = _NCH // _GCH
_GS = _GCH * _CHUNK


def _norm_k(q):
    var = jnp.mean(q * q, axis=-1, keepdims=True)
    return q * lax.rsqrt(var + 1e-6) * (1.0 / 8.0)


def _attn_body(qs_ref, idx_ref, qh_ref, ih_ref, out_ref):
    qv = qs_ref[0]
    q = qv[:, :_DH]
    v = qv[:, _DH:]
    k = _norm_k(q)
    qvh = qh_ref[0]
    kh = _norm_k(qvh[:, :_DH])
    q3 = q.reshape(_GCH, _CHUNK, _DH)
    k3 = k.reshape(_GCH, _CHUNK, _DH)
    v3 = v.reshape(_GCH, _CHUNK, _DH)
    kp = jnp.concatenate([kh[None], k3[:-1]], axis=0)
    vp = jnp.concatenate([qvh[None, :, _DH:], v3[:-1]], axis=0)
    qi = idx_ref[0]
    kip = jnp.concatenate([ih_ref[0], qi[:-1]], axis=0)

    def dots(a, b):
        return lax.dot_general(a, b, (((2,), (2,)), ((0,), (0,))),
                               preferred_element_type=jnp.float32)

    def mask(d, kidx):
        ql = qi[:, :, None]
        kl = kidx[:, None, :]
        d = jnp.where(ql >= kl, d, jnp.float32(-1e9))
        return jnp.where(ql != kl, d, jnp.float32(-1e5))

    dp = mask(dots(q3, kp), kip)
    ds = mask(dots(q3, k3), qi)
    m = jnp.maximum(jnp.max(dp, axis=-1), jnp.max(ds, axis=-1))
    ep = jnp.exp(dp - m[:, :, None])
    es = jnp.exp(ds - m[:, :, None])
    ssum = jnp.sum(ep, axis=-1) + jnp.sum(es, axis=-1)
    lse = m + jnp.log(ssum)

    def pv(p, vv):
        return lax.dot_general(p, vv, (((2,), (1,)), ((0,), (0,))),
                               preferred_element_type=jnp.float32)
    o = (pv(ep, vp) + pv(es, v3)) / ssum[:, :, None]
    lse_bc = jnp.broadcast_to(lse[:, :, None], (_GCH, _CHUNK, _DH))
    pk = jnp.concatenate([o, lse_bc], axis=2)
    out_ref[0] = pk.reshape(_GS, 2 * _DH)


def _attention(qsv, sbi3):
    hn = qsv.shape[0]
    def prev(h, g):
        return (h, (g * _GCH - 1) % _NCH, 0)
    return pl.pallas_call(
        _attn_body,
        grid=(hn, _NG),
        in_specs=[
            pl.BlockSpec((1, _GS, 2 * _DH), lambda h, g: (h, g, 0)),
            pl.BlockSpec((1, _GCH, _CHUNK), lambda h, g: (h, g, 0)),
            pl.BlockSpec((1, _CHUNK, 2 * _DH), prev),
            pl.BlockSpec((1, 1, _CHUNK),
                         lambda h, g: (h * _NCH + (g * _GCH - 1) % _NCH, 0, 0)),
        ],
        out_specs=pl.BlockSpec((1, _GS, 2 * _DH), lambda h, g: (h, g, 0)),
        out_shape=jax.ShapeDtypeStruct((hn, _NHR * _S, 2 * _DH), jnp.float32),
        interpret=_INTERPRET,
    )(qsv, sbi3, qsv, sbi3.reshape(hn * _NCH, 1, _CHUNK))


def _comb_body(og1_ref, og2_ref, out_ref):
    for h in range(_H):
        ref = og1_ref if h < _H // 2 else og2_ref
        hl = h % (_H // 2)
        b0 = ref[2 * hl]
        b1 = ref[2 * hl + 1]
        l0 = b0[:, _DH:_DH + 1]
        l1 = b1[:, _DH:_DH + 1]
        m = jnp.maximum(l0, l1)
        e0 = jnp.exp(l0 - m)
        e1 = jnp.exp(l1 - m)
        t = e0 + e1
        o = b0[:, :_DH] * (e0 / t) + b1[:, :_DH] * (e1 / t)
        out_ref[:, h * _DH:(h + 1) * _DH] = o


def _combine(og1, og2):
    grid = _S // _SB
    hseg = _SEG // 2
    return pl.pallas_call(
        _comb_body,
        grid=(grid,),
        in_specs=[
            pl.BlockSpec((hseg, _SB, 2 * _DH), lambda i: (0, i, 0)),
            pl.BlockSpec((hseg, _SB, 2 * _DH), lambda i: (0, i, 0)),
        ],
        out_specs=pl.BlockSpec((_SB, _H * _DH), lambda i: (i, 0)),
        out_shape=jax.ShapeDtypeStruct((_S, _H * _DH), jnp.float32),
        interpret=_INTERPRET,
    )(og1, og2)


_CK = 128
_NCK = _S // _CK


_HG = _H // 2
_SEGG = _SEG // 2


def _make_gather_body(h_off):
    def _gather_qkv_body(dest_ref, qkv_ref,
                         sbi_out_ref, qs_ref,
                         dv, sbiv, gidxv, rowa, rowb, sem, sem2):
        wid = lax.axis_index("s") * 2 + lax.axis_index("c")

        @pl.when(wid < _SEGG)
        def _():
            hl = wid // _NHR
            r = lax.rem(wid, _NHR)
            pltpu.sync_copy(dest_ref.at[wid], dv)
            i16 = lax.iota(jnp.int32, 16)

            def scat(i, _):
                idx = dv[pl.ds(i * 16, 16)]
                tok = i * 16 + i16
                plsc.store_scatter(sbiv, [idx], tok)
                plsc.store_scatter(gidxv, [idx], tok * _H + (hl + h_off))
                return 0
            lax.fori_loop(0, _S // 16, scat, 0, unroll=8)

            pltpu.sync_copy(sbiv, sbi_out_ref.at[hl, pl.ds(r * _S, _S)])

            def gat(j2, _):
                j = j2 * 2
                ca = pltpu.async_copy(qkv_ref.at[gidxv.at[pl.ds(j * _CK, _CK)]],
                                      rowa, sem)
                cb = pltpu.async_copy(
                    qkv_ref.at[gidxv.at[pl.ds((j + 1) * _CK, _CK)]],
                    rowb, sem2)
                ca.wait()
                pltpu.sync_copy(rowa, qs_ref.at[hl, pl.ds(r * _S + j * _CK, _CK)])
                cb.wait()
                pltpu.sync_copy(rowb,
                                qs_ref.at[hl, pl.ds(r * _S + (j + 1) * _CK, _CK)])
                return 0
            lax.fori_loop(0, _NCK // 2, gat, 0)
    return _gather_qkv_body


def _gather_qkv(dest, qkv_t, h_off):
    mesh = plsc.VectorSubcoreMesh(core_axis_name="c", subcore_axis_name="s")
    k = functools.partial(
        pl.kernel,
        out_type=[
            jax.ShapeDtypeStruct((_HG, _NHR * _S), jnp.int32),
            jax.ShapeDtypeStruct((_HG, _NHR * _S, 2 * _DH), jnp.float32),
        ],
        mesh=mesh,
        compiler_params=pltpu.CompilerParams(needs_layout_passes=False, use_tc_tiling_on_sc=False),
        scratch_types=[
            pltpu.VMEM((_S,), jnp.int32),
            pltpu.VMEM((_S,), jnp.int32),
            pltpu.VMEM((_S,), jnp.int32),
            pltpu.VMEM((_CK, 2 * _DH), jnp.float32),
            pltpu.VMEM((_CK, 2 * _DH), jnp.float32),
            pltpu.SemaphoreType.DMA,
            pltpu.SemaphoreType.DMA,
        ],
    )(_make_gather_body(h_off))
    return k(dest, qkv_t)


def _unsort_body(dest_ref, outs_ref, og_ref,
                 dv, gidxv, rowa, rowb, sem, sem2):
    wid = lax.axis_index("s") * 2 + lax.axis_index("c")

    @pl.when(wid < _SEGG)
    def _():
        h = wid // _NHR
        r = lax.rem(wid, _NHR)
        pltpu.sync_copy(dest_ref.at[wid], dv)
        base = h * (_NHR * _S) + r * _S

        def mk(i, _):
            gidxv[pl.ds(i * 16, 16)] = dv[pl.ds(i * 16, 16)] + base
            return 0
        lax.fori_loop(0, _S // 16, mk, 0, unroll=8)

        def gat(j2, _):
            j = j2 * 2
            ca = pltpu.async_copy(outs_ref.at[gidxv.at[pl.ds(j * _CK, _CK)]],
                                  rowa, sem)
            cb = pltpu.async_copy(outs_ref.at[gidxv.at[pl.ds((j + 1) * _CK, _CK)]],
                                  rowb, sem2)
            ca.wait()
            pltpu.sync_copy(rowa, og_ref.at[wid, pl.ds(j * _CK, _CK)])
            cb.wait()
            pltpu.sync_copy(rowb, og_ref.at[wid, pl.ds((j + 1) * _CK, _CK)])
            return 0
        lax.fori_loop(0, _NCK // 2, gat, 0)


def _unsort(dest, out_s2):
    mesh = plsc.VectorSubcoreMesh(core_axis_name="c", subcore_axis_name="s")
    k = functools.partial(
        pl.kernel,
        out_type=jax.ShapeDtypeStruct((_SEGG, _S, 2 * _DH), jnp.float32),
        mesh=mesh,
        compiler_params=pltpu.CompilerParams(needs_layout_passes=False, use_tc_tiling_on_sc=False),
        scratch_types=[
            pltpu.VMEM((_S,), jnp.int32),
            pltpu.VMEM((_S,), jnp.int32),
            pltpu.VMEM((_CK, 2 * _DH), jnp.float32),
            pltpu.VMEM((_CK, 2 * _DH), jnp.float32),
            pltpu.SemaphoreType.DMA,
            pltpu.SemaphoreType.DMA,
        ],
    )(_unsort_body)
    return k(dest, out_s2)


def kernel(hidden_states, Wqk, Wv, rotations):
    h2 = hidden_states[0]
    rot2 = rotations.reshape(_DH, _NHR * 128)

    qkv, bkt = _proj_hash(h2, Wqk, Wv, rot2)
    dest = _sort_dest(bkt).reshape(_SEG, _S)

    qkv_t = qkv.reshape(_S * _H, 2 * _DH)

    sbi1, qsv1 = _gather_qkv(dest[:_SEGG], qkv_t, 0)
    sbi2, qsv2 = _gather_qkv(dest[_SEGG:], qkv_t, _HG)
    out1 = _attention(qsv1, sbi1.reshape(_HG, _NCH, _CHUNK))
    out2 = _attention(qsv2, sbi2.reshape(_HG, _NCH, _CHUNK))
    og1 = _unsort(dest[:_SEGG], out1.reshape(_HG * _NHR * _S, 2 * _DH))
    og2 = _unsort(dest[_SEGG:], out2.reshape(_HG * _NHR * _S, 2 * _DH))
    final = _combine(og1, og2)
    return final.reshape(1, _S, _H * _DH)


# speedup vs baseline: 1.0716x; 1.0716x over previous
import functools

import jax
import jax.numpy as jnp
from jax import lax
from jax.experimental import pallas as pl
from jax.experimental.pallas import tpu as pltpu
from jax.experimental.pallas import tpu_sc as plsc

_S, _D = 8192, 768
_H, _DH = 12, 64
_NHR = 2
_NB = 256
_CHUNK = 64
_NCH = _S * _NHR // _CHUNK
_SEG = _H * _NHR
_SB = 512

_INTERPRET = False


def _proj_hash_body(h_ref, wqk_ref, wv_ref, rot_ref, eye_ref, qkv_ref, bkt_ref):
    hb = h_ref[...]
    wqk = wqk_ref[...]
    wv = wv_ref[...]
    eye = eye_ref[...]
    qk = lax.dot_general(hb, wqk, (((1,), (1,)), ((), ())),
                         preferred_element_type=jnp.float32)
    v = lax.dot_general(hb, wv, (((1,), (1,)), ((), ())),
                        preferred_element_type=jnp.float32)
    pieces = []
    rot = rot_ref[...]
    i2 = lax.broadcasted_iota(jnp.int32, (_SB, 128), 1)
    for h in range(_H):
        qh = qk[:, h * _DH:(h + 1) * _DH]
        pieces.append(qh)
        pieces.append(v[:, h * _DH:(h + 1) * _DH])
        r = lax.dot_general(qh, rot, (((1,), (0,)), ((), ())),
                            preferred_element_type=jnp.float32)
        for n in range(_NHR):
            rr = r[:, n * 128:(n + 1) * 128]
            mx = jnp.max(jnp.abs(rr), axis=1, keepdims=True)
            t1 = jnp.where(rr == mx, i2, 2 * _NB)
            t2 = jnp.where(rr == -mx, i2 + 128, 2 * _NB)
            am = jnp.min(jnp.minimum(t1, t2), axis=1,
                         keepdims=True).astype(jnp.float32)
            am_row = lax.dot_general(am, eye, (((0,), (0,)), ((), ())),
                                     preferred_element_type=jnp.float32)
            c = h * _NHR + n
            bkt_ref[c:c + 1, :] = am_row
    qkv_ref[...] = jnp.concatenate(pieces, axis=1)


def _proj_hash(h2, wqk, wv, rot2):
    grid = _S // _SB
    eye = jnp.eye(_SB, dtype=jnp.float32)
    return pl.pallas_call(
        _proj_hash_body,
        grid=(grid,),
        in_specs=[
            pl.BlockSpec((_SB, _D), lambda i: (i, 0)),
            pl.BlockSpec((_H * _DH, _D), lambda i: (0, 0)),
            pl.BlockSpec((_H * _DH, _D), lambda i: (0, 0)),
            pl.BlockSpec((_DH, _NHR * 128), lambda i: (0, 0)),
            pl.BlockSpec((_SB, _SB), lambda i: (0, 0)),
        ],
        out_specs=[
            pl.BlockSpec((_SB, _H * 2 * _DH), lambda i: (i, 0)),
            pl.BlockSpec((_SEG, _SB), lambda i: (0, i)),
        ],
        out_shape=[
            jax.ShapeDtypeStruct((_S, _H * 2 * _DH), jnp.float32),
            jax.ShapeDtypeStruct((_SEG, _S), jnp.float32),
        ],
        interpret=_INTERPRET,
    )(h2, wqk, wv, rot2, eye)


_NT = _S // 128


def _sort_body(bkt_ref, tri_ref, low_ref, dest_ref):
    triu = tri_ref[...]
    low = low_ref[...]

    beta = lax.broadcasted_iota(jnp.int32, (_SEG, _NB, 128), 1)

    def onehot(t):
        b = bkt_ref[:, pl.ds(t * 128, 128)].astype(jnp.int32)
        return (b[:, None, :] == beta).astype(jnp.float32)

    def p1_body(t, carry):
        return carry + jnp.sum(onehot(t), axis=2)
    tot = lax.fori_loop(0, _NT, p1_body,
                        jnp.zeros((_SEG, _NB), jnp.float32), unroll=4)

    start = lax.dot_general(tot, low, (((1,), (0,)), ((), ())),
                            preferred_element_type=jnp.float32)

    def p2_body(t, pre):
        oh3 = onehot(t)
        oh2 = oh3.reshape(_SEG * _NB, 128)
        cnt = lax.dot_general(oh2, triu, (((1,), (0,)), ((), ())),
                              preferred_element_type=jnp.float32)
        cnt3 = cnt.reshape(_SEG, _NB, 128)
        d3 = oh3 * (cnt3 + (pre + start)[:, :, None])
        dest = jnp.sum(d3, axis=1) - 1.0
        dest_ref[:, :, pl.ds(t * 128, 128)] = dest[:, None, :].astype(jnp.int32)
        return pre + jnp.sum(oh3, axis=2)
    lax.fori_loop(0, _NT, p2_body, jnp.zeros((_SEG, _NB), jnp.float32),
                  unroll=4)


def _sort_dest(bkt):
    r = lax.broadcasted_iota(jnp.int32, (128, 128), 0)
    c = lax.broadcasted_iota(jnp.int32, (128, 128), 1)
    triu = (r <= c).astype(jnp.float32)
    r2 = lax.broadcasted_iota(jnp.int32, (_NB, _NB), 0)
    c2 = lax.broadcasted_iota(jnp.int32, (_NB, _NB), 1)
    low = (r2 < c2).astype(jnp.float32)
    return pl.pallas_call(
        _sort_body,
        grid=(1,),
        in_specs=[pl.BlockSpec((_SEG, _S), lambda i: (0, 0)),
                  pl.BlockSpec((128, 128), lambda i: (0, 0)),
                  pl.BlockSpec((_NB, _NB), lambda i: (0, 0))],
        out_specs=pl.BlockSpec((_SEG, 1, _S), lambda i: (0, 0, 0)),
        out_shape=jax.ShapeDtypeStruct((_SEG, 1, _S), jnp.int32),
        interpret=_INTERPRET,
    )(bkt, triu, low)


_GCH = 32
_NG = _NCH // _GCH
_GS = _GCH * _CHUNK


def _norm_k(q):
    var = jnp.mean(q * q, axis=-1, keepdims=True)
    return q * lax.rsqrt(var + 1e-6) * (1.0 / 8.0)


def _attn_body(qs_ref, idx_ref, qh_ref, ih_ref, out_ref):
    qv = qs_ref[0]
    q = qv[:, :_DH]
    v = qv[:, _DH:]
    k = _norm_k(q)
    qvh = qh_ref[0]
    kh = _norm_k(qvh[:, :_DH])
    q3 = q.reshape(_GCH, _CHUNK, _DH)
    k3 = k.reshape(_GCH, _CHUNK, _DH)
    v3 = v.reshape(_GCH, _CHUNK, _DH)
    kp = jnp.concatenate([kh[None], k3[:-1]], axis=0)
    vp = jnp.concatenate([qvh[None, :, _DH:], v3[:-1]], axis=0)
    qi = idx_ref[0]
    kip = jnp.concatenate([ih_ref[0], qi[:-1]], axis=0)

    def dots(a, b):
        return lax.dot_general(a, b, (((2,), (2,)), ((0,), (0,))),
                               preferred_element_type=jnp.float32)

    def mask(d, kidx):
        ql = qi[:, :, None]
        kl = kidx[:, None, :]
        d = jnp.where(ql >= kl, d, jnp.float32(-1e9))
        return jnp.where(ql != kl, d, jnp.float32(-1e5))

    dp = mask(dots(q3, kp), kip)
    ds = mask(dots(q3, k3), qi)
    m = jnp.maximum(jnp.max(dp, axis=-1), jnp.max(ds, axis=-1))
    ep = jnp.exp(dp - m[:, :, None])
    es = jnp.exp(ds - m[:, :, None])
    ssum = jnp.sum(ep, axis=-1) + jnp.sum(es, axis=-1)
    lse = m + jnp.log(ssum)

    def pv(p, vv):
        return lax.dot_general(p, vv, (((2,), (1,)), ((0,), (0,))),
                               preferred_element_type=jnp.float32)
    o = (pv(ep, vp) + pv(es, v3)) / ssum[:, :, None]
    lse_bc = jnp.broadcast_to(lse[:, :, None], (_GCH, _CHUNK, _DH))
    pk = jnp.concatenate([o, lse_bc], axis=2)
    out_ref[0] = pk.reshape(_GS, 2 * _DH)


def _attention(qsv, sbi3, h_off, hn):
    def prev(h, g):
        return (h_off + h, (g * _GCH - 1) % _NCH, 0)
    return pl.pallas_call(
        _attn_body,
        grid=(hn, _NG),
        in_specs=[
            pl.BlockSpec((1, _GS, 2 * _DH), lambda h, g: (h_off + h, g, 0)),
            pl.BlockSpec((1, _GCH, _CHUNK), lambda h, g: (h_off + h, g, 0)),
            pl.BlockSpec((1, _CHUNK, 2 * _DH), prev),
            pl.BlockSpec((1, 1, _CHUNK),
                         lambda h, g: ((h_off + h) * _NCH
                                       + (g * _GCH - 1) % _NCH, 0, 0)),
        ],
        out_specs=pl.BlockSpec((1, _GS, 2 * _DH), lambda h, g: (h, g, 0)),
        out_shape=jax.ShapeDtypeStruct((hn, _NHR * _S, 2 * _DH), jnp.float32),
        interpret=_INTERPRET,
    )(qsv, sbi3, qsv, sbi3.reshape(_H * _NCH, 1, _CHUNK))


def _comb_body(og1_ref, og2_ref, out_ref):
    for h in range(_H):
        ref = og1_ref if h < _H // 2 else og2_ref
        hl = h % (_H // 2)
        b0 = ref[2 * hl]
        b1 = ref[2 * hl + 1]
        l0 = b0[:, _DH:_DH + 1]
        l1 = b1[:, _DH:_DH + 1]
        m = jnp.maximum(l0, l1)
        e0 = jnp.exp(l0 - m)
        e1 = jnp.exp(l1 - m)
        t = e0 + e1
        o = b0[:, :_DH] * (e0 / t) + b1[:, :_DH] * (e1 / t)
        out_ref[:, h * _DH:(h + 1) * _DH] = o


def _combine(og1, og2):
    grid = _S // _SB
    hseg = _SEG // 2
    return pl.pallas_call(
        _comb_body,
        grid=(grid,),
        in_specs=[
            pl.BlockSpec((hseg, _SB, 2 * _DH), lambda i: (0, i, 0)),
            pl.BlockSpec((hseg, _SB, 2 * _DH), lambda i: (0, i, 0)),
        ],
        out_specs=pl.BlockSpec((_SB, _H * _DH), lambda i: (i, 0)),
        out_shape=jax.ShapeDtypeStruct((_S, _H * _DH), jnp.float32),
        interpret=_INTERPRET,
    )(og1, og2)


_CK = 128
_NCK = _S // _CK


_HG = _H // 2
_SEGG = _SEG // 2


def _gather_qkv_body(dest_ref, qkv_ref,
                     sbi_out_ref, qs_ref,
                     dv, sbiv, gidxv, rowa, rowb, sem, sem2):
    wid = lax.axis_index("s") * 2 + lax.axis_index("c")

    @pl.when(wid < _SEG)
    def _():
        h = wid // _NHR
        r = lax.rem(wid, _NHR)
        pltpu.sync_copy(dest_ref.at[wid], dv)
        i16 = lax.iota(jnp.int32, 16)

        def scat(i, _):
            idx = dv[pl.ds(i * 16, 16)]
            tok = i * 16 + i16
            plsc.store_scatter(sbiv, [idx], tok)
            plsc.store_scatter(gidxv, [idx], tok * _H + h)
            return 0
        lax.fori_loop(0, _S // 16, scat, 0, unroll=8)

        pltpu.sync_copy(sbiv, sbi_out_ref.at[h, pl.ds(r * _S, _S)])

        def gat(j2, _):
            j = j2 * 2
            ca = pltpu.async_copy(qkv_ref.at[gidxv.at[pl.ds(j * _CK, _CK)]],
                                  rowa, sem)
            cb = pltpu.async_copy(
                qkv_ref.at[gidxv.at[pl.ds((j + 1) * _CK, _CK)]],
                rowb, sem2)
            ca.wait()
            pltpu.sync_copy(rowa, qs_ref.at[h, pl.ds(r * _S + j * _CK, _CK)])
            cb.wait()
            pltpu.sync_copy(rowb,
                            qs_ref.at[h, pl.ds(r * _S + (j + 1) * _CK, _CK)])
            return 0
        lax.fori_loop(0, _NCK // 2, gat, 0)


def _gather_qkv(dest, qkv_t):
    mesh = plsc.VectorSubcoreMesh(core_axis_name="c", subcore_axis_name="s")
    k = functools.partial(
        pl.kernel,
        out_type=[
            jax.ShapeDtypeStruct((_H, _NHR * _S), jnp.int32),
            jax.ShapeDtypeStruct((_H, _NHR * _S, 2 * _DH), jnp.float32),
        ],
        mesh=mesh,
        compiler_params=pltpu.CompilerParams(needs_layout_passes=False, use_tc_tiling_on_sc=False),
        scratch_types=[
            pltpu.VMEM((_S,), jnp.int32),
            pltpu.VMEM((_S,), jnp.int32),
            pltpu.VMEM((_S,), jnp.int32),
            pltpu.VMEM((_CK, 2 * _DH), jnp.float32),
            pltpu.VMEM((_CK, 2 * _DH), jnp.float32),
            pltpu.SemaphoreType.DMA,
            pltpu.SemaphoreType.DMA,
        ],
    )(_gather_qkv_body)
    return k(dest, qkv_t)


_FR = _SEGG * _S // 32
_FCK = _FR // _CK


def _unsort_body(dest_ref, outs_ref, og_ref,
                 dv, gidxv, rowa, rowb, sem, sem2):
    wid = lax.axis_index("s") * 2 + lax.axis_index("c")
    start = wid * _FR
    pltpu.sync_copy(dest_ref.at[pl.ds(start, _FR)], dv)
    i16 = lax.iota(jnp.int32, 16)

    def mk(i, _):
        u = start + i * 16 + i16
        gidxv[pl.ds(i * 16, 16)] = (dv[pl.ds(i * 16, 16)]
                                    + (u & jnp.int32(~(_S - 1))))
        return 0
    lax.fori_loop(0, _FR // 16, mk, 0, unroll=8)

    def gat(j2, _):
        j = j2 * 2
        ca = pltpu.async_copy(outs_ref.at[gidxv.at[pl.ds(j * _CK, _CK)]],
                              rowa, sem)
        cb = pltpu.async_copy(outs_ref.at[gidxv.at[pl.ds((j + 1) * _CK, _CK)]],
                              rowb, sem2)
        ca.wait()
        pltpu.sync_copy(rowa, og_ref.at[pl.ds(start + j * _CK, _CK)])
        cb.wait()
        pltpu.sync_copy(rowb, og_ref.at[pl.ds(start + (j + 1) * _CK, _CK)])
        return 0
    lax.fori_loop(0, _FCK // 2, gat, 0)


def _unsort(dest_flat, out_s2):
    mesh = plsc.VectorSubcoreMesh(core_axis_name="c", subcore_axis_name="s")
    k = functools.partial(
        pl.kernel,
        out_type=jax.ShapeDtypeStruct((_SEGG * _S, 2 * _DH), jnp.float32),
        mesh=mesh,
        compiler_params=pltpu.CompilerParams(needs_layout_passes=False, use_tc_tiling_on_sc=False),
        scratch_types=[
            pltpu.VMEM((_FR,), jnp.int32),
            pltpu.VMEM((_FR,), jnp.int32),
            pltpu.VMEM((_CK, 2 * _DH), jnp.float32),
            pltpu.VMEM((_CK, 2 * _DH), jnp.float32),
            pltpu.SemaphoreType.DMA,
            pltpu.SemaphoreType.DMA,
        ],
    )(_unsort_body)
    return k(dest_flat, out_s2)


def kernel(hidden_states, Wqk, Wv, rotations):
    h2 = hidden_states[0]
    rot2 = rotations.reshape(_DH, _NHR * 128)

    qkv, bkt = _proj_hash(h2, Wqk, Wv, rot2)
    dest = _sort_dest(bkt).reshape(_SEG, _S)

    qkv_t = qkv.reshape(_S * _H, 2 * _DH)

    sbi, qsv = _gather_qkv(dest, qkv_t)
    sbi3 = sbi.reshape(_H, _NCH, _CHUNK)

    out1 = _attention(qsv, sbi3, 0, _HG)
    out2 = _attention(qsv, sbi3, _HG, _HG)
    og1 = _unsort(dest[:_SEGG].reshape(_SEGG * _S),
                  out1.reshape(_HG * _NHR * _S, 2 * _DH))
    og2 = _unsort(dest[_SEGG:].reshape(_SEGG * _S),
                  out2.reshape(_HG * _NHR * _S, 2 * _DH))
    final = _combine(og1.reshape(_SEGG, _S, 2 * _DH),
                     og2.reshape(_SEGG, _S, 2 * _DH))
    return final.reshape(1, _S, _H * _DH)


# speedup vs baseline: 1.0867x; 1.0141x over previous
import functools

import jax
import jax.numpy as jnp
from jax import lax
from jax.experimental import pallas as pl
from jax.experimental.pallas import tpu as pltpu
from jax.experimental.pallas import tpu_sc as plsc

_S, _D = 8192, 768
_H, _DH = 12, 64
_NHR = 2
_NB = 256
_CHUNK = 64
_NCH = _S * _NHR // _CHUNK
_SEG = _H * _NHR
_SB = 512

_INTERPRET = False


def _proj_hash_body(h_ref, wqk_ref, wv_ref, rot_ref, eye_ref, qkv_ref, bkt_ref):
    hb = h_ref[...]
    wqk = wqk_ref[...]
    wv = wv_ref[...]
    eye = eye_ref[...]
    qk = lax.dot_general(hb, wqk, (((1,), (1,)), ((), ())),
                         preferred_element_type=jnp.float32)
    v = lax.dot_general(hb, wv, (((1,), (1,)), ((), ())),
                        preferred_element_type=jnp.float32)
    pieces = []
    rot = rot_ref[...]
    i2 = lax.broadcasted_iota(jnp.int32, (_SB, 128), 1)
    for h in range(_H):
        qh = qk[:, h * _DH:(h + 1) * _DH]
        pieces.append(qh)
        pieces.append(v[:, h * _DH:(h + 1) * _DH])
        r = lax.dot_general(qh, rot, (((1,), (0,)), ((), ())),
                            preferred_element_type=jnp.float32)
        for n in range(_NHR):
            rr = r[:, n * 128:(n + 1) * 128]
            mx = jnp.max(jnp.abs(rr), axis=1, keepdims=True)
            t1 = jnp.where(rr == mx, i2, 2 * _NB)
            t2 = jnp.where(rr == -mx, i2 + 128, 2 * _NB)
            am = jnp.min(jnp.minimum(t1, t2), axis=1,
                         keepdims=True).astype(jnp.float32)
            am_row = lax.dot_general(am, eye, (((0,), (0,)), ((), ())),
                                     preferred_element_type=jnp.float32)
            c = h * _NHR + n
            bkt_ref[c:c + 1, :] = am_row
    qkv_ref[...] = jnp.concatenate(pieces, axis=1)


def _proj_hash(h2, wqk, wv, rot2):
    grid = _S // _SB
    eye = jnp.eye(_SB, dtype=jnp.float32)
    return pl.pallas_call(
        _proj_hash_body,
        grid=(grid,),
        in_specs=[
            pl.BlockSpec((_SB, _D), lambda i: (i, 0)),
            pl.BlockSpec((_H * _DH, _D), lambda i: (0, 0)),
            pl.BlockSpec((_H * _DH, _D), lambda i: (0, 0)),
            pl.BlockSpec((_DH, _NHR * 128), lambda i: (0, 0)),
            pl.BlockSpec((_SB, _SB), lambda i: (0, 0)),
        ],
        out_specs=[
            pl.BlockSpec((_SB, _H * 2 * _DH), lambda i: (i, 0)),
            pl.BlockSpec((_SEG, _SB), lambda i: (0, i)),
        ],
        out_shape=[
            jax.ShapeDtypeStruct((_S, _H * 2 * _DH), jnp.float32),
            jax.ShapeDtypeStruct((_SEG, _S), jnp.float32),
        ],
        interpret=_INTERPRET,
    )(h2, wqk, wv, rot2, eye)


_NT = _S // 128


def _sort_body(bkt_ref, tri_ref, low_ref, dest_ref):
    triu = tri_ref[...]
    low = low_ref[...]

    beta = lax.broadcasted_iota(jnp.int32, (_SEG, _NB, 128), 1)

    def onehot(t):
        b = bkt_ref[:, pl.ds(t * 128, 128)].astype(jnp.int32)
        return (b[:, None, :] == beta).astype(jnp.float32)

    def p1_body(t, carry):
        return carry + jnp.sum(onehot(t), axis=2)
    tot = lax.fori_loop(0, _NT, p1_body,
                        jnp.zeros((_SEG, _NB), jnp.float32), unroll=4)

    start = lax.dot_general(tot, low, (((1,), (0,)), ((), ())),
                            preferred_element_type=jnp.float32)

    def p2_body(t, pre):
        oh3 = onehot(t)
        oh2 = oh3.reshape(_SEG * _NB, 128)
        cnt = lax.dot_general(oh2, triu, (((1,), (0,)), ((), ())),
                              preferred_element_type=jnp.float32)
        cnt3 = cnt.reshape(_SEG, _NB, 128)
        d3 = oh3 * (cnt3 + (pre + start)[:, :, None])
        dest = jnp.sum(d3, axis=1) - 1.0
        dest_ref[:, :, pl.ds(t * 128, 128)] = dest[:, None, :].astype(jnp.int32)
        return pre + jnp.sum(oh3, axis=2)
    lax.fori_loop(0, _NT, p2_body, jnp.zeros((_SEG, _NB), jnp.float32),
                  unroll=4)


def _sort_dest(bkt):
    r = lax.broadcasted_iota(jnp.int32, (128, 128), 0)
    c = lax.broadcasted_iota(jnp.int32, (128, 128), 1)
    triu = (r <= c).astype(jnp.float32)
    r2 = lax.broadcasted_iota(jnp.int32, (_NB, _NB), 0)
    c2 = lax.broadcasted_iota(jnp.int32, (_NB, _NB), 1)
    low = (r2 < c2).astype(jnp.float32)
    return pl.pallas_call(
        _sort_body,
        grid=(1,),
        in_specs=[pl.BlockSpec((_SEG, _S), lambda i: (0, 0)),
                  pl.BlockSpec((128, 128), lambda i: (0, 0)),
                  pl.BlockSpec((_NB, _NB), lambda i: (0, 0))],
        out_specs=pl.BlockSpec((_SEG, 1, _S), lambda i: (0, 0, 0)),
        out_shape=jax.ShapeDtypeStruct((_SEG, 1, _S), jnp.int32),
        interpret=_INTERPRET,
    )(bkt, triu, low)


_GCH = 32
_NG = _NCH // _GCH
_GS = _GCH * _CHUNK


def _norm_k(q):
    var = jnp.mean(q * q, axis=-1, keepdims=True)
    return q * lax.rsqrt(var + 1e-6) * (1.0 / 8.0)


def _attn_body(qs_ref, idx_ref, qh_ref, ih_ref, out_ref):
    qv = qs_ref[0]
    q = qv[:, :_DH]
    v = qv[:, _DH:]
    k = _norm_k(q)
    qvh = qh_ref[0]
    kh = _norm_k(qvh[:, :_DH])
    q3 = q.reshape(_GCH, _CHUNK, _DH)
    k3 = k.reshape(_GCH, _CHUNK, _DH)
    v3 = v.reshape(_GCH, _CHUNK, _DH)
    kp = jnp.concatenate([kh[None], k3[:-1]], axis=0)
    vp = jnp.concatenate([qvh[None, :, _DH:], v3[:-1]], axis=0)
    qi = idx_ref[0]
    kip = jnp.concatenate([ih_ref[0], qi[:-1]], axis=0)

    def dots(a, b):
        return lax.dot_general(a, b, (((2,), (2,)), ((0,), (0,))),
                               preferred_element_type=jnp.float32)

    def mask(d, kidx):
        ql = qi[:, :, None]
        kl = kidx[:, None, :]
        d = jnp.where(ql >= kl, d, jnp.float32(-1e9))
        return jnp.where(ql != kl, d, jnp.float32(-1e5))

    dp = mask(dots(q3, kp), kip)
    ds = mask(dots(q3, k3), qi)
    m = jnp.maximum(jnp.max(dp, axis=-1), jnp.max(ds, axis=-1))
    ep = jnp.exp(dp - m[:, :, None])
    es = jnp.exp(ds - m[:, :, None])
    ssum = jnp.sum(ep, axis=-1) + jnp.sum(es, axis=-1)
    lse = m + jnp.log(ssum)

    def pv(p, vv):
        return lax.dot_general(p, vv, (((2,), (1,)), ((0,), (0,))),
                               preferred_element_type=jnp.float32)
    o = (pv(ep, vp) + pv(es, v3)) / ssum[:, :, None]
    lse_bc = jnp.broadcast_to(lse[:, :, None], (_GCH, _CHUNK, _DH))
    pk = jnp.concatenate([o, lse_bc], axis=2)
    out_ref[0] = pk.reshape(_GS, 2 * _DH)


def _attention(qsv, sbi3, h_off, hn):
    def prev(h, g):
        return (h_off + h, (g * _GCH - 1) % _NCH, 0)
    return pl.pallas_call(
        _attn_body,
        grid=(hn, _NG),
        in_specs=[
            pl.BlockSpec((1, _GS, 2 * _DH), lambda h, g: (h_off + h, g, 0)),
            pl.BlockSpec((1, _GCH, _CHUNK), lambda h, g: (h_off + h, g, 0)),
            pl.BlockSpec((1, _CHUNK, 2 * _DH), prev),
            pl.BlockSpec((1, 1, _CHUNK),
                         lambda h, g: ((h_off + h) * _NCH
                                       + (g * _GCH - 1) % _NCH, 0, 0)),
        ],
        out_specs=pl.BlockSpec((1, _GS, 2 * _DH), lambda h, g: (h, g, 0)),
        out_shape=jax.ShapeDtypeStruct((hn, _NHR * _S, 2 * _DH), jnp.float32),
        interpret=_INTERPRET,
    )(qsv, sbi3, qsv, sbi3.reshape(_H * _NCH, 1, _CHUNK))


def _comb_body(og1_ref, og2_ref, out_ref):
    for h in range(_H):
        ref = og1_ref if h < _H // 2 else og2_ref
        hl = h % (_H // 2)
        b0 = ref[2 * hl]
        b1 = ref[2 * hl + 1]
        l0 = b0[:, _DH:_DH + 1]
        l1 = b1[:, _DH:_DH + 1]
        m = jnp.maximum(l0, l1)
        e0 = jnp.exp(l0 - m)
        e1 = jnp.exp(l1 - m)
        t = e0 + e1
        o = b0[:, :_DH] * (e0 / t) + b1[:, :_DH] * (e1 / t)
        out_ref[:, h * _DH:(h + 1) * _DH] = o


def _combine(og1, og2):
    grid = _S // _SB
    hseg = _SEG // 2
    return pl.pallas_call(
        _comb_body,
        grid=(grid,),
        in_specs=[
            pl.BlockSpec((hseg, _SB, 2 * _DH), lambda i: (0, i, 0)),
            pl.BlockSpec((hseg, _SB, 2 * _DH), lambda i: (0, i, 0)),
        ],
        out_specs=pl.BlockSpec((_SB, _H * _DH), lambda i: (i, 0)),
        out_shape=jax.ShapeDtypeStruct((_S, _H * _DH), jnp.float32),
        interpret=_INTERPRET,
    )(og1, og2)


_CK = 128
_NCK = _S // _CK


_HG = _H // 2
_SEGG = _SEG // 2


def _gather_qkv_body(dest_ref, qkv_ref,
                     sbi_out_ref, qs_ref,
                     dv, sbiv, gidxv, r0, r1, r2, r3,
                     g0, g1, g2, g3, o0, o1, o2, o3):
    rows = (r0, r1, r2, r3)
    gsems = (g0, g1, g2, g3)
    osems = (o0, o1, o2, o3)
    wid = lax.axis_index("s") * 2 + lax.axis_index("c")

    @pl.when(wid < _SEG)
    def _():
        h = wid // _NHR
        r = lax.rem(wid, _NHR)
        pltpu.sync_copy(dest_ref.at[wid], dv)
        i16 = lax.iota(jnp.int32, 16)

        def scat(i, _):
            idx = dv[pl.ds(i * 16, 16)]
            tok = i * 16 + i16
            plsc.store_scatter(sbiv, [idx], tok)
            plsc.store_scatter(gidxv, [idx], tok * _H + h)
            return 0
        lax.fori_loop(0, _S // 16, scat, 0, unroll=8)

        pltpu.sync_copy(sbiv, sbi_out_ref.at[h, pl.ds(r * _S, _S)])

        def gat(j4, _):
            j = j4 * 4
            cs = [pltpu.async_copy(
                      qkv_ref.at[gidxv.at[pl.ds((j + i) * _CK, _CK)]],
                      rows[i], gsems[i]) for i in range(4)]
            outs = []
            for i in range(4):
                cs[i].wait()
                outs.append(pltpu.async_copy(
                    rows[i], qs_ref.at[h, pl.ds(r * _S + (j + i) * _CK, _CK)],
                    osems[i]))
            for o in outs:
                o.wait()
            return 0
        lax.fori_loop(0, _NCK // 4, gat, 0)


def _gather_qkv(dest, qkv_t):
    mesh = plsc.VectorSubcoreMesh(core_axis_name="c", subcore_axis_name="s")
    k = functools.partial(
        pl.kernel,
        out_type=[
            jax.ShapeDtypeStruct((_H, _NHR * _S), jnp.int32),
            jax.ShapeDtypeStruct((_H, _NHR * _S, 2 * _DH), jnp.float32),
        ],
        mesh=mesh,
        compiler_params=pltpu.CompilerParams(needs_layout_passes=False, use_tc_tiling_on_sc=False),
        scratch_types=[
            pltpu.VMEM((_S,), jnp.int32),
            pltpu.VMEM((_S,), jnp.int32),
            pltpu.VMEM((_S,), jnp.int32),
            pltpu.VMEM((_CK, 2 * _DH), jnp.float32),
            pltpu.VMEM((_CK, 2 * _DH), jnp.float32),
            pltpu.VMEM((_CK, 2 * _DH), jnp.float32),
            pltpu.VMEM((_CK, 2 * _DH), jnp.float32),
        ] + [pltpu.SemaphoreType.DMA] * 8,
    )(_gather_qkv_body)
    return k(dest, qkv_t)


_FR = _SEGG * _S // 32
_FCK = _FR // _CK


def _unsort_body(dest_ref, outs_ref, og_ref,
                 dv, gidxv, r0, r1, r2, r3,
                 g0, g1, g2, g3, o0, o1, o2, o3):
    rows = (r0, r1, r2, r3)
    gsems = (g0, g1, g2, g3)
    osems = (o0, o1, o2, o3)
    wid = lax.axis_index("s") * 2 + lax.axis_index("c")
    start = wid * _FR
    pltpu.sync_copy(dest_ref.at[pl.ds(start, _FR)], dv)
    i16 = lax.iota(jnp.int32, 16)

    def mk(i, _):
        u = start + i * 16 + i16
        gidxv[pl.ds(i * 16, 16)] = (dv[pl.ds(i * 16, 16)]
                                    + (u & jnp.int32(~(_S - 1))))
        return 0
    lax.fori_loop(0, _FR // 16, mk, 0, unroll=8)

    def gat(j4, _):
        j = j4 * 4
        cs = [pltpu.async_copy(
                  outs_ref.at[gidxv.at[pl.ds((j + i) * _CK, _CK)]],
                  rows[i], gsems[i]) for i in range(4)]
        outs = []
        for i in range(4):
            cs[i].wait()
            outs.append(pltpu.async_copy(
                rows[i], og_ref.at[pl.ds(start + (j + i) * _CK, _CK)],
                osems[i]))
        for o in outs:
            o.wait()
        return 0
    lax.fori_loop(0, _FCK // 4, gat, 0)


def _unsort(dest_flat, out_s2):
    mesh = plsc.VectorSubcoreMesh(core_axis_name="c", subcore_axis_name="s")
    k = functools.partial(
        pl.kernel,
        out_type=jax.ShapeDtypeStruct((_SEGG * _S, 2 * _DH), jnp.float32),
        mesh=mesh,
        compiler_params=pltpu.CompilerParams(needs_layout_passes=False, use_tc_tiling_on_sc=False),
        scratch_types=[
            pltpu.VMEM((_FR,), jnp.int32),
            pltpu.VMEM((_FR,), jnp.int32),
            pltpu.VMEM((_CK, 2 * _DH), jnp.float32),
            pltpu.VMEM((_CK, 2 * _DH), jnp.float32),
            pltpu.VMEM((_CK, 2 * _DH), jnp.float32),
            pltpu.VMEM((_CK, 2 * _DH), jnp.float32),
        ] + [pltpu.SemaphoreType.DMA] * 8,
    )(_unsort_body)
    return k(dest_flat, out_s2)


def kernel(hidden_states, Wqk, Wv, rotations):
    h2 = hidden_states[0]
    rot2 = rotations.reshape(_DH, _NHR * 128)

    qkv, bkt = _proj_hash(h2, Wqk, Wv, rot2)
    dest = _sort_dest(bkt).reshape(_SEG, _S)

    qkv_t = qkv.reshape(_S * _H, 2 * _DH)

    sbi, qsv = _gather_qkv(dest, qkv_t)
    sbi3 = sbi.reshape(_H, _NCH, _CHUNK)

    out1 = _attention(qsv, sbi3, 0, _HG)
    out2 = _attention(qsv, sbi3, _HG, _HG)
    og1 = _unsort(dest[:_SEGG].reshape(_SEGG * _S),
                  out1.reshape(_HG * _NHR * _S, 2 * _DH))
    og2 = _unsort(dest[_SEGG:].reshape(_SEGG * _S),
                  out2.reshape(_HG * _NHR * _S, 2 * _DH))
    final = _combine(og1.reshape(_SEGG, _S, 2 * _DH),
                     og2.reshape(_SEGG, _S, 2 * _DH))
    return final.reshape(1, _S, _H * _DH)
